# baseline (speedup 1.0000x reference)
import functools

import jax
import jax.numpy as jnp
from jax import lax
from jax.experimental import pallas as pl
from jax.experimental.pallas import tpu as pltpu
from jax.experimental.pallas import tpu_sc as plsc

EMB = 16
N_EDGES = 1600000
CHUNK = 1280
NUM_CHUNKS = N_EDGES // CHUNK
NW = 32
NBUF = 2


def _sc_lookup(ci, Wc_rep):
    mesh = plsc.VectorSubcoreMesh(core_axis_name="c", subcore_axis_name="s")

    @functools.partial(
        pl.kernel,
        mesh=mesh,
        compiler_params=pltpu.CompilerParams(
            use_tc_tiling_on_sc=False, needs_layout_passes=False),
        out_type=jax.ShapeDtypeStruct((N_EDGES, 2 * EMB), jnp.float32),
        scratch_types=[
            [pltpu.VMEM((CHUNK,), jnp.int32) for _ in range(NBUF)],
            [pltpu.VMEM((CHUNK, 2 * EMB), jnp.float32) for _ in range(NBUF)],
            [pltpu.SemaphoreType.DMA for _ in range(NBUF)],
            [pltpu.SemaphoreType.DMA for _ in range(NBUF)],
            [pltpu.SemaphoreType.DMA for _ in range(NBUF)],
            pltpu.VMEM_SHARED((16, 2 * EMB), jnp.float32),
        ],
    )
    def k(ci_hbm, wc_hbm, out_hbm, ci_v, out_v, isem, gsem, wsem, wc_sh):
        wid = lax.axis_index("s") * 2 + lax.axis_index("c")
        steps = (NUM_CHUNKS + NW - 1) // NW
        tw = (NUM_CHUNKS - wid + NW - 1) // NW

        @pl.when(lax.axis_index("s") == 0)
        def _():
            pltpu.sync_copy(wc_hbm.at[0], wc_sh)
        plsc.subcore_barrier()
        my_wc = wc_sh

        def start_idx(t, b):
            base = (wid + t * NW) * CHUNK
            pltpu.async_copy(ci_hbm.at[pl.ds(base, CHUNK)], ci_v[b], isem[b])

        def wait_idx(b):
            pltpu.make_async_copy(
                ci_hbm.at[pl.ds(0, CHUNK)], ci_v[b], isem[b]).wait()

        def wait_write(b):
            pltpu.make_async_copy(
                out_v[b], out_hbm.at[pl.ds(0, CHUNK), :], wsem[b]).wait()

        def run_chunk(t, b):
            wait_idx(b)
            cps = []
            for j in range(0, CHUNK, 128):
                cps.append(pltpu.async_copy(
                    my_wc.at[ci_v[b].at[pl.ds(j, 128)]],
                    out_v[b].at[pl.ds(j, 128), :], gsem[b]))
            for cp in cps:
                cp.wait()
            base = (wid + t * NW) * CHUNK
            pltpu.async_copy(out_v[b], out_hbm.at[pl.ds(base, CHUNK), :], wsem[b])

        start_idx(0, 0)

        def body(t, carry):
            for bb in range(NBUF):
                @pl.when(lax.rem(t, NBUF) == bb)
                def _(bb=bb):
                    @pl.when(t + 1 < tw)
                    def _():
                        start_idx(t + 1, (bb + 1) % NBUF)

                    @pl.when(t < tw)
                    def _():
                        @pl.when(t >= NBUF)
                        def _():
                            wait_write(bb)
                        run_chunk(t, bb)
            return carry

        lax.fori_loop(0, steps, body, 0)

        for kk in range(NBUF):
            tp = tw - 1 - kk
            for bb in range(NBUF):
                @pl.when(jnp.logical_and(tp >= 0, lax.rem(tp, NBUF) == bb))
                def _(bb=bb):
                    wait_write(bb)

    return k(ci, Wc_rep)


def kernel(edge_attr, W0, W1):
    Wc = jnp.concatenate(
        [jnp.repeat(W0, 4, axis=0), jnp.tile(W1, (4, 1))], axis=1)
    Wc_rep = jnp.tile(Wc[None], (NW, 1, 1))
    ci = jnp.dot(edge_attr.astype(jnp.float32),
                 jnp.array([4.0, 1.0], jnp.float32)).astype(jnp.int32)
    return _sc_lookup(ci, Wc_rep)


# speedup vs baseline: 1.0488x; 1.0488x over previous
import functools

import jax
import jax.numpy as jnp
from jax import lax
from jax.experimental import pallas as pl
from jax.experimental.pallas import tpu as pltpu
from jax.experimental.pallas import tpu_sc as plsc

EMB = 16
N_EDGES = 1600000
CHUNK = 1280
NUM_CHUNKS = N_EDGES // CHUNK
NW = 32
L = 16
NBUF = 2


def _sc_lookup(eaT, Wc):
    mesh = plsc.VectorSubcoreMesh(core_axis_name="c", subcore_axis_name="s")

    @functools.partial(
        pl.kernel,
        mesh=mesh,
        compiler_params=pltpu.CompilerParams(
            use_tc_tiling_on_sc=False, needs_layout_passes=False),
        out_type=jax.ShapeDtypeStruct((N_EDGES, 2 * EMB), jnp.float32),
        scratch_types=[
            [pltpu.VMEM((CHUNK,), jnp.int32) for _ in range(NBUF)],
            [pltpu.VMEM((CHUNK,), jnp.int32) for _ in range(NBUF)],
            [pltpu.VMEM((CHUNK,), jnp.int32) for _ in range(NBUF)],
            [pltpu.VMEM((CHUNK, 2 * EMB), jnp.float32) for _ in range(NBUF)],
            [pltpu.SemaphoreType.DMA for _ in range(NBUF)],
            [pltpu.SemaphoreType.DMA for _ in range(NBUF)],
            [pltpu.SemaphoreType.DMA for _ in range(NBUF)],
            pltpu.VMEM_SHARED((16, 2 * EMB), jnp.float32),
        ],
    )
    def k(ea_hbm, wc_hbm, out_hbm,
          i0_v, i1_v, ci_v, out_v, isem, gsem, wsem, wc_sh):
        wid = lax.axis_index("s") * 2 + lax.axis_index("c")
        steps = (NUM_CHUNKS + NW - 1) // NW
        tw = (NUM_CHUNKS - wid + NW - 1) // NW

        @pl.when(lax.axis_index("s") == 0)
        def _():
            pltpu.sync_copy(wc_hbm, wc_sh)
        plsc.subcore_barrier()

        def start_idx(t, b):
            base = (wid + t * NW) * CHUNK
            pltpu.async_copy(ea_hbm.at[0, pl.ds(base, CHUNK)], i0_v[b], isem[b])
            pltpu.async_copy(ea_hbm.at[1, pl.ds(base, CHUNK)], i1_v[b], isem[b])

        def wait_idx(b):
            pltpu.make_async_copy(
                ea_hbm.at[0, pl.ds(0, CHUNK)], i0_v[b], isem[b]).wait()
            pltpu.make_async_copy(
                ea_hbm.at[1, pl.ds(0, CHUNK)], i1_v[b], isem[b]).wait()

        def wait_write(b):
            pltpu.make_async_copy(
                out_v[b], out_hbm.at[pl.ds(0, CHUNK), :], wsem[b]).wait()

        def run_chunk(t, b):
            wait_idx(b)
            for o in range(0, CHUNK, L):
                ci_v[b][pl.ds(o, L)] = (
                    i0_v[b][pl.ds(o, L)] * 4 + i1_v[b][pl.ds(o, L)])
            cps = []
            for j in range(0, CHUNK, 128):
                cps.append(pltpu.async_copy(
                    wc_sh.at[ci_v[b].at[pl.ds(j, 128)]],
                    out_v[b].at[pl.ds(j, 128), :], gsem[b]))
            for cp in cps:
                cp.wait()
            base = (wid + t * NW) * CHUNK
            pltpu.async_copy(out_v[b], out_hbm.at[pl.ds(base, CHUNK), :], wsem[b])

        start_idx(0, 0)

        def body(t, carry):
            for bb in range(NBUF):
                @pl.when(lax.rem(t, NBUF) == bb)
                def _(bb=bb):
                    @pl.when(t + 1 < tw)
                    def _():
                        start_idx(t + 1, (bb + 1) % NBUF)

                    @pl.when(t < tw)
                    def _():
                        @pl.when(t >= NBUF)
                        def _():
                            wait_write(bb)
                        run_chunk(t, bb)
            return carry

        lax.fori_loop(0, steps, body, 0)

        for kk in range(NBUF):
            tp = tw - 1 - kk
            for bb in range(NBUF):
                @pl.when(jnp.logical_and(tp >= 0, lax.rem(tp, NBUF) == bb))
                def _(bb=bb):
                    wait_write(bb)

    return k(eaT, Wc)


def kernel(edge_attr, W0, W1):
    Wc = jnp.concatenate(
        [jnp.repeat(W0, 4, axis=0), jnp.tile(W1, (4, 1))], axis=1)
    return _sc_lookup(edge_attr.T, Wc)


# speedup vs baseline: 5.7238x; 5.4574x over previous
import functools

import jax
import jax.numpy as jnp
from jax import lax
from jax.experimental import pallas as pl
from jax.experimental.pallas import tpu as pltpu
from jax.experimental.pallas import tpu_sc as plsc

EMB = 16
N_EDGES = 1600000
CHUNK = 1280
NUM_CHUNKS = N_EDGES // CHUNK
NW = 32
L = 16
NBUF = 2


def _sc_lookup(ci, WcT):
    mesh = plsc.VectorSubcoreMesh(core_axis_name="c", subcore_axis_name="s")

    @functools.partial(
        pl.kernel,
        mesh=mesh,
        compiler_params=pltpu.CompilerParams(
            use_tc_tiling_on_sc=True, needs_layout_passes=False),
        out_type=jax.ShapeDtypeStruct((2 * EMB, N_EDGES), jnp.float32),
        scratch_types=[
            [pltpu.VMEM((CHUNK,), jnp.int32) for _ in range(NBUF)],
            [pltpu.VMEM((2 * EMB, CHUNK), jnp.float32) for _ in range(NBUF)],
            pltpu.VMEM((2 * EMB, EMB), jnp.float32),
            [pltpu.SemaphoreType.DMA for _ in range(NBUF)],
            [pltpu.SemaphoreType.DMA for _ in range(NBUF)],
        ],
    )
    def k(ci_hbm, wct_hbm, out_hbm, ci_v, out_v, wct_v, isem, wsem):
        wid = lax.axis_index("s") * 2 + lax.axis_index("c")
        steps = (NUM_CHUNKS + NW - 1) // NW
        tw = (NUM_CHUNKS - wid + NW - 1) // NW

        pltpu.sync_copy(wct_hbm, wct_v)
        wcols = [wct_v[f, :] for f in range(2 * EMB)]

        def start_idx(t, b):
            base = (wid + t * NW) * CHUNK
            pltpu.async_copy(ci_hbm.at[pl.ds(base, CHUNK)], ci_v[b], isem[b])

        def wait_idx(b):
            pltpu.make_async_copy(
                ci_hbm.at[pl.ds(0, CHUNK)], ci_v[b], isem[b]).wait()

        def wait_write(b):
            pltpu.make_async_copy(
                out_v[b], out_hbm.at[:, pl.ds(0, CHUNK)], wsem[b]).wait()

        def run_chunk(t, b):
            wait_idx(b)

            def transpose_group(o, carry):
                ov = o * L
                civ = ci_v[b][pl.ds(ov, L)]
                for f in range(2 * EMB):
                    out_v[b][f, pl.ds(ov, L)] = (
                        wcols[f].at[civ].get(mode="promise_in_bounds"))
                return carry

            lax.fori_loop(0, CHUNK // L, transpose_group, 0)
            base = (wid + t * NW) * CHUNK
            pltpu.async_copy(out_v[b], out_hbm.at[:, pl.ds(base, CHUNK)], wsem[b])

        start_idx(0, 0)

        def body(t, carry):
            for bb in range(NBUF):
                @pl.when(lax.rem(t, NBUF) == bb)
                def _(bb=bb):
                    @pl.when(t + 1 < tw)
                    def _():
                        start_idx(t + 1, (bb + 1) % NBUF)

                    @pl.when(t < tw)
                    def _():
                        @pl.when(t >= NBUF)
                        def _():
                            wait_write(bb)
                        run_chunk(t, bb)
            return carry

        lax.fori_loop(0, steps, body, 0)

        for kk in range(NBUF):
            tp = tw - 1 - kk
            for bb in range(NBUF):
                @pl.when(jnp.logical_and(tp >= 0, lax.rem(tp, NBUF) == bb))
                def _(bb=bb):
                    wait_write(bb)

    return k(ci, WcT)


def kernel(edge_attr, W0, W1):
    Wc = jnp.concatenate(
        [jnp.repeat(W0, 4, axis=0), jnp.tile(W1, (4, 1))], axis=1)
    ci = edge_attr[:, 0] * 4 + edge_attr[:, 1]
    res_t = _sc_lookup(ci, Wc.T)
    return res_t.T


# speedup vs baseline: 7.2327x; 1.2636x over previous
import functools

import jax
import jax.numpy as jnp
from jax import lax
from jax.experimental import pallas as pl
from jax.experimental.pallas import tpu as pltpu
from jax.experimental.pallas import tpu_sc as plsc

EMB = 16
N_EDGES = 1600000
CHUNK = 1024
ROWS = CHUNK // 128
TAIL = 512
NUM_MAIN = (N_EDGES - TAIL) // CHUNK
NW = 32
L = 16
NBUF = 2


def _sc_lookup(ci2d, ci_tail, WcT):
    mesh = plsc.VectorSubcoreMesh(core_axis_name="c", subcore_axis_name="s")

    @functools.partial(
        pl.kernel,
        mesh=mesh,
        compiler_params=pltpu.CompilerParams(
            use_tc_tiling_on_sc=True, needs_layout_passes=False),
        out_type=jax.ShapeDtypeStruct((2 * EMB, N_EDGES), jnp.float32),
        scratch_types=[
            [pltpu.VMEM((ROWS, 128), jnp.int32) for _ in range(NBUF)],
            [pltpu.VMEM((2 * EMB, CHUNK), jnp.float32) for _ in range(NBUF)],
            pltpu.VMEM((TAIL,), jnp.int32),
            pltpu.VMEM((2 * EMB, EMB), jnp.float32),
            [pltpu.SemaphoreType.DMA for _ in range(NBUF)],
            [pltpu.SemaphoreType.DMA for _ in range(NBUF)],
        ],
    )
    def k(ci_hbm, tail_hbm, wct_hbm, out_hbm,
          ci_v, out_v, tail_v, wct_v, isem, wsem):
        wid = lax.axis_index("s") * 2 + lax.axis_index("c")
        steps = (NUM_MAIN + NW - 1) // NW
        tw = (NUM_MAIN - wid + NW - 1) // NW

        pltpu.sync_copy(wct_hbm, wct_v)
        wcols = [wct_v[f, :] for f in range(2 * EMB)]

        def start_idx(t, b):
            r0 = (wid + t * NW) * ROWS
            pltpu.async_copy(ci_hbm.at[pl.ds(r0, ROWS), :], ci_v[b], isem[b])

        def wait_idx(b):
            pltpu.make_async_copy(
                ci_hbm.at[pl.ds(0, ROWS), :], ci_v[b], isem[b]).wait()

        def wait_write(b):
            pltpu.make_async_copy(
                out_v[b], out_hbm.at[:, pl.ds(0, CHUNK)], wsem[b]).wait()

        def transpose_groups(idx_ref, dst_ref, n_groups):
            def group(o, carry):
                ov = o * L
                civ = idx_ref[pl.ds(ov, L)]
                for f in range(2 * EMB):
                    dst_ref[f, pl.ds(ov, L)] = (
                        wcols[f].at[civ].get(mode="promise_in_bounds"))
                return carry

            lax.fori_loop(0, n_groups, group, 0)

        def run_chunk(t, b):
            wait_idx(b)

            def group(o, carry):
                r = o // (128 // L)
                ov = (o % (128 // L)) * L
                civ = ci_v[b][r, pl.ds(ov, L)]
                for f in range(2 * EMB):
                    out_v[b][f, pl.ds(o * L, L)] = (
                        wcols[f].at[civ].get(mode="promise_in_bounds"))
                return carry

            lax.fori_loop(0, CHUNK // L, group, 0)
            base = (wid + t * NW) * CHUNK
            pltpu.async_copy(out_v[b], out_hbm.at[:, pl.ds(base, CHUNK)], wsem[b])

        start_idx(0, 0)

        def body(t, carry):
            for bb in range(NBUF):
                @pl.when(lax.rem(t, NBUF) == bb)
                def _(bb=bb):
                    @pl.when(t + 1 < tw)
                    def _():
                        start_idx(t + 1, (bb + 1) % NBUF)

                    @pl.when(t < tw)
                    def _():
                        @pl.when(t >= NBUF)
                        def _():
                            wait_write(bb)
                        run_chunk(t, bb)
            return carry

        lax.fori_loop(0, steps, body, 0)

        for kk in range(NBUF):
            tp = tw - 1 - kk
            for bb in range(NBUF):
                @pl.when(jnp.logical_and(tp >= 0, lax.rem(tp, NBUF) == bb))
                def _(bb=bb):
                    wait_write(bb)

        @pl.when(wid == NW - 1)
        def _():
            pltpu.sync_copy(tail_hbm, tail_v)

            def group(o, carry):
                ov = o * L
                civ = tail_v[pl.ds(ov, L)]
                for f in range(2 * EMB):
                    out_v[0][f, pl.ds(ov, L)] = (
                        wcols[f].at[civ].get(mode="promise_in_bounds"))
                return carry

            lax.fori_loop(0, TAIL // L, group, 0)
            pltpu.sync_copy(
                out_v[0].at[:, pl.ds(0, TAIL)],
                out_hbm.at[:, pl.ds(NUM_MAIN * CHUNK, TAIL)])

    return k(ci2d, ci_tail, WcT)


def kernel(edge_attr, W0, W1):
    Wc = jnp.concatenate(
        [jnp.repeat(W0, 4, axis=0), jnp.tile(W1, (4, 1))], axis=1)
    ea_t = edge_attr.reshape(N_EDGES // 128, 128, 2).transpose(0, 2, 1)
    ci2d = ea_t[:, 0, :] * 4 + ea_t[:, 1, :]
    tail0 = NUM_MAIN * CHUNK
    ci_tail = edge_attr[tail0:, 0] * 4 + edge_attr[tail0:, 1]
    res_t = _sc_lookup(ci2d, ci_tail, Wc.T)
    return res_t.T
